# baseline (speedup 1.0000x reference)
import functools

import jax
import jax.numpy as jnp
from jax import lax
from jax.experimental import pallas as pl
from jax.experimental.pallas import tpu as pltpu
from jax.experimental.pallas import tpu_sc as plsc

T = 2048
D = 1024
H = 2048
E = 8
K = 2
BLK = 256
P = 6144
NBLK = P // BLK


def _dot_bf16(a, b):
    return lax.dot_general(a.astype(jnp.bfloat16), b.astype(jnp.bfloat16),
                           (((1,), (0,)), ((), ())),
                           preferred_element_type=jnp.float32)


def _dot_default(a, b):
    return lax.dot_general(a, b, (((1,), (0,)), ((), ())),
                           precision=jax.lax.Precision.DEFAULT,
                           preferred_element_type=jnp.float32)


def _router_dispatch_body(x_ref, rw_ref, rb_ref,
                          tok_ref, wt_ref, be_ref, act_ref, posc_ref):
    x = x_ref[...]
    W = rw_ref[...]
    b = rb_ref[...]
    logits = _dot_default(x, W) + b

    iota_e = lax.broadcasted_iota(jnp.int32, (1, E), 1).astype(jnp.float32)
    big = jnp.float32(E)
    m1 = jnp.max(logits, axis=1, keepdims=True)
    a0 = jnp.min(jnp.where(logits == m1, iota_e, big), axis=1, keepdims=True)
    masked = jnp.where(iota_e == a0, -jnp.inf, logits)
    m2 = jnp.max(masked, axis=1, keepdims=True)
    a1 = jnp.min(jnp.where(masked == m2, iota_e, big), axis=1, keepdims=True)
    w0 = 1.0 / (1.0 + jnp.exp(m2 - m1))
    w1 = 1.0 - w0

    OH0 = (iota_e == a0).astype(jnp.float32)
    OH1 = (iota_e == a1).astype(jnp.float32)

    r_i = lax.broadcasted_iota(jnp.int32, (T, T), 0)
    c_i = lax.broadcasted_iota(jnp.int32, (T, T), 1)
    L = (c_i < r_i).astype(jnp.bfloat16)
    OH01 = jnp.concatenate([OH0, OH1], axis=1)
    C01 = _dot_bf16(L, OH01)
    C0, C1 = C01[:, :E], C01[:, E:]
    cnt0 = jnp.sum(OH0, axis=0, keepdims=True)
    cnt1 = jnp.sum(OH1, axis=0, keepdims=True)
    counts = cnt0 + cnt1
    pc = jnp.floor((counts + (BLK - 1.0)) * (1.0 / BLK)) * BLK
    f_i = lax.broadcasted_iota(jnp.int32, (E, E), 0)
    e_i = lax.broadcasted_iota(jnp.int32, (E, E), 1)
    M8 = (f_i < e_i).astype(jnp.float32)
    starts = _dot_bf16(pc, M8)

    pos0 = jnp.sum((C0 + starts) * OH0, axis=1, keepdims=True)
    pos1 = jnp.sum((C1 + starts + cnt0) * OH1, axis=1, keepdims=True)
    posc_ref[...] = jnp.concatenate([pos0, pos1], axis=1).astype(jnp.int32)

    bidx = lax.broadcasted_iota(jnp.int32, (NBLK, E), 0).astype(jnp.float32) * jnp.float32(BLK)
    be = jnp.sum((starts <= bidx).astype(jnp.float32), axis=1, keepdims=True)
    be_ref[...] = (be - 1.0).astype(jnp.int32)
    total = jnp.sum(pc)
    act_ref[...] = (bidx[:, 0:1] < total).astype(jnp.int32)

    tokf = lax.broadcasted_iota(jnp.int32, (T, 1), 0).astype(jnp.float32)
    tok_hi = jnp.floor(tokf * (1.0 / 32.0))
    tok_lo = tokf - tok_hi * 32.0
    tv0 = jnp.concatenate([tok_hi, tok_lo, w0], axis=1)
    tv1 = jnp.concatenate([tok_hi, tok_lo, w1], axis=1)
    CH = 512
    for q in range(P // CH):
        tq = lax.broadcasted_iota(jnp.int32, (1, CH), 1).astype(jnp.float32) + jnp.float32(CH * q)
        M0 = (pos0 == tq).astype(jnp.float32)
        M1 = (pos1 == tq).astype(jnp.float32)
        acc = (lax.dot_general(M0.astype(jnp.bfloat16), tv0.astype(jnp.bfloat16),
                               (((0,), (0,)), ((), ())),
                               preferred_element_type=jnp.float32)
               + lax.dot_general(M1.astype(jnp.bfloat16), tv1.astype(jnp.bfloat16),
                                 (((0,), (0,)), ((), ())),
                                 preferred_element_type=jnp.float32))
        tok_ref[pl.ds(q * CH, CH), :] = (acc[:, 0:1] * 32.0 + acc[:, 1:2]).astype(jnp.int32)
        wt_ref[pl.ds(q * CH, CH), :] = acc[:, 2:3]


def _router_dispatch(x, router_W, router_b):
    return pl.pallas_call(
        _router_dispatch_body,
        out_shape=(
            jax.ShapeDtypeStruct((P, 1), jnp.int32),
            jax.ShapeDtypeStruct((P, 1), jnp.float32),
            jax.ShapeDtypeStruct((NBLK, 1), jnp.int32),
            jax.ShapeDtypeStruct((NBLK, 1), jnp.int32),
            jax.ShapeDtypeStruct((T, 2), jnp.int32),
        ),
    )(x, router_W, router_b.reshape(1, E))


def _gelu_exact(h):
    return 0.5 * h * (1.0 + lax.erf(h * jnp.float32(0.7071067811865476)))


def _mlp_body(be_ref, act_ref, xbf_ref, tok_ref, w1_ref, b1_ref, w2_ref,
              b2_ref, wt_ref, o_ref):
    g = pl.program_id(0)

    @pl.when(act_ref[g] != 0)
    def _():
        tok_blk = tok_ref[...]
        iota_t = lax.broadcasted_iota(jnp.int32, (BLK, T), 1)
        OH = (tok_blk == iota_t).astype(jnp.bfloat16)
        xg = _dot_bf16(OH, xbf_ref[...])
        h = _dot_bf16(xg, w1_ref[...][0]) + b1_ref[...][0]
        h = _gelu_exact(h)
        o = _dot_bf16(h, w2_ref[...][0]) + b2_ref[...][0]
        o_bf = o.astype(jnp.bfloat16).astype(jnp.float32)
        wt_bf = wt_ref[...].astype(jnp.bfloat16).astype(jnp.float32)
        o_ref[...] = o_bf * wt_bf


def _grouped_mlp(x, tok, w1, b1, w2, b2, wt, be, act):
    grid_spec = pltpu.PrefetchScalarGridSpec(
        num_scalar_prefetch=2,
        grid=(NBLK,),
        in_specs=[
            pl.BlockSpec((T, D), lambda g, be, act: (0, 0)),
            pl.BlockSpec((BLK, 1), lambda g, be, act: (g, 0)),
            pl.BlockSpec((1, D, H), lambda g, be, act: (be[g], 0, 0)),
            pl.BlockSpec((1, 1, H), lambda g, be, act: (be[g], 0, 0)),
            pl.BlockSpec((1, H, D), lambda g, be, act: (be[g], 0, 0)),
            pl.BlockSpec((1, 1, D), lambda g, be, act: (be[g], 0, 0)),
            pl.BlockSpec((BLK, 1), lambda g, be, act: (g, 0)),
        ],
        out_specs=pl.BlockSpec((BLK, D), lambda g, be, act: (g, 0)),
    )
    return pl.pallas_call(
        _mlp_body,
        grid_spec=grid_spec,
        out_shape=jax.ShapeDtypeStruct((P, D), jnp.float32),
    )(be, act, x, tok, w1, b1.reshape(E, 1, H), w2, b2.reshape(E, 1, D), wt)


def _sc_combine(o, posc):
    NC, NS = 2, 16
    NW = NC * NS
    t_per_w = T // NW
    CT = 16
    NCH = t_per_w // CT
    mesh = plsc.VectorSubcoreMesh(core_axis_name="c", subcore_axis_name="s")

    @functools.partial(
        pl.kernel, mesh=mesh,
        out_type=jax.ShapeDtypeStruct((T, D), jnp.float32),
        scratch_types=[
            pltpu.VMEM((NCH, 2 * CT), jnp.int32),
            pltpu.VMEM((2 * CT, D), jnp.float32),
            pltpu.VMEM((2 * CT, D), jnp.float32),
            pltpu.VMEM((CT, D), jnp.float32),
            pltpu.VMEM((CT, D), jnp.float32),
            pltpu.SemaphoreType.DMA,
            pltpu.SemaphoreType.DMA,
            pltpu.SemaphoreType.DMA,
        ],
    )
    def k(o_hbm, pc_hbm, out_hbm, idx2, rc0, rc1, ob0, ob1, s0, s1, sw):
        wid = lax.axis_index("s") * NC + lax.axis_index("c")
        base_t = wid * t_per_w
        base_i = base_t * 2
        for c in range(NCH):
            pltpu.sync_copy(pc_hbm.at[pl.ds(base_i + c * 2 * CT, 2 * CT)],
                            idx2.at[c])
        rcs = (rc0, rc1)
        obs = (ob0, ob1)
        sems = (s0, s1)
        gets = [None] * NCH
        puts = [None] * NCH
        gets[0] = pltpu.async_copy(o_hbm.at[idx2.at[0]], rc0, s0)
        for c in range(NCH):
            if c + 1 < NCH:
                gets[c + 1] = pltpu.async_copy(o_hbm.at[idx2.at[c + 1]],
                                               rcs[(c + 1) % 2],
                                               sems[(c + 1) % 2])
            gets[c].wait()
            rc = rcs[c % 2]
            ob = obs[c % 2]
            if c >= 2:
                puts[c - 2].wait()
            for r in range(CT):
                def add_vec(j, _, r=r, rc=rc, ob=ob):
                    ob[r, pl.ds(j * 16, 16)] = (rc[2 * r, pl.ds(j * 16, 16)]
                                                + rc[2 * r + 1, pl.ds(j * 16, 16)])
                    return 0
                lax.fori_loop(0, D // 16, add_vec, 0, unroll=8)
            puts[c] = pltpu.async_copy(
                ob, out_hbm.at[pl.ds(base_t + c * CT, CT)], sw)
        puts[NCH - 2].wait()
        puts[NCH - 1].wait()

    return k(o, posc)


def kernel(x, router_W, router_b, w1, b1, w2, b2):
    tok, wt, be, act, posc = _router_dispatch(x, router_W, router_b)
    o = _grouped_mlp(x.astype(jnp.bfloat16), tok, w1.astype(jnp.bfloat16), b1,
                     w2.astype(jnp.bfloat16), b2, wt,
                     be.reshape(NBLK), act.reshape(NBLK))
    return _sc_combine(o, posc.reshape(2 * T))


# speedup vs baseline: 1.2444x; 1.2444x over previous
import functools

import jax
import jax.numpy as jnp
from jax import lax
from jax.experimental import pallas as pl
from jax.experimental.pallas import tpu as pltpu
from jax.experimental.pallas import tpu_sc as plsc

T = 2048
D = 1024
H = 2048
E = 8
K = 2
BLK = 256
P = 6144
NBLK = P // BLK


def _dot_bf16(a, b):
    return lax.dot_general(a.astype(jnp.bfloat16), b.astype(jnp.bfloat16),
                           (((1,), (0,)), ((), ())),
                           preferred_element_type=jnp.float32)


def _dot_default(a, b):
    return lax.dot_general(a, b, (((1,), (0,)), ((), ())),
                           precision=jax.lax.Precision.DEFAULT,
                           preferred_element_type=jnp.float32)


def _router_dispatch_body(x_ref, rw_ref, rb_ref,
                          tok_ref, wt_ref, be_ref, act_ref, posc_ref):
    x = x_ref[...]
    W = rw_ref[...]
    b = rb_ref[...]
    logits = _dot_default(x, W) + b

    iota_e = lax.broadcasted_iota(jnp.int32, (1, E), 1).astype(jnp.float32)
    big = jnp.float32(E)
    m1 = jnp.max(logits, axis=1, keepdims=True)
    a0 = jnp.min(jnp.where(logits == m1, iota_e, big), axis=1, keepdims=True)
    masked = jnp.where(iota_e == a0, -jnp.inf, logits)
    m2 = jnp.max(masked, axis=1, keepdims=True)
    a1 = jnp.min(jnp.where(masked == m2, iota_e, big), axis=1, keepdims=True)
    w0 = 1.0 / (1.0 + jnp.exp(m2 - m1))
    w1 = 1.0 - w0

    OH0 = (iota_e == a0).astype(jnp.float32)
    OH1 = (iota_e == a1).astype(jnp.float32)

    r_i = lax.broadcasted_iota(jnp.int32, (T, T), 0)
    c_i = lax.broadcasted_iota(jnp.int32, (T, T), 1)
    L = (c_i < r_i).astype(jnp.bfloat16)
    OH01 = jnp.concatenate([OH0, OH1], axis=1)
    C01 = _dot_bf16(L, OH01)
    C0, C1 = C01[:, :E], C01[:, E:]
    cnt0 = jnp.sum(OH0, axis=0, keepdims=True)
    cnt1 = jnp.sum(OH1, axis=0, keepdims=True)
    counts = cnt0 + cnt1
    pc = jnp.floor((counts + (BLK - 1.0)) * (1.0 / BLK)) * BLK
    f_i = lax.broadcasted_iota(jnp.int32, (E, E), 0)
    e_i = lax.broadcasted_iota(jnp.int32, (E, E), 1)
    M8 = (f_i < e_i).astype(jnp.float32)
    starts = _dot_bf16(pc, M8)

    pos0 = jnp.sum((C0 + starts) * OH0, axis=1, keepdims=True)
    pos1 = jnp.sum((C1 + starts + cnt0) * OH1, axis=1, keepdims=True)
    posc_ref[...] = jnp.concatenate([pos0, pos1], axis=1).astype(jnp.int32)

    bidx = lax.broadcasted_iota(jnp.int32, (NBLK, E), 0).astype(jnp.float32) * jnp.float32(BLK)
    be = jnp.sum((starts <= bidx).astype(jnp.float32), axis=1, keepdims=True)
    be_ref[...] = (be - 1.0).astype(jnp.int32)
    total = jnp.sum(pc)
    act_ref[...] = (bidx[:, 0:1] < total).astype(jnp.int32)

    tokf = lax.broadcasted_iota(jnp.int32, (T, 1), 0).astype(jnp.float32)
    tok_hi = jnp.floor(tokf * (1.0 / 32.0))
    tok_lo = tokf - tok_hi * 32.0
    tv0 = jnp.concatenate([tok_hi, tok_lo, w0], axis=1)
    tv1 = jnp.concatenate([tok_hi, tok_lo, w1], axis=1)
    CH = 512
    for q in range(P // CH):
        tq = lax.broadcasted_iota(jnp.int32, (1, CH), 1).astype(jnp.float32) + jnp.float32(CH * q)
        M0 = (pos0 == tq).astype(jnp.float32)
        M1 = (pos1 == tq).astype(jnp.float32)
        acc = (lax.dot_general(M0.astype(jnp.bfloat16), tv0.astype(jnp.bfloat16),
                               (((0,), (0,)), ((), ())),
                               preferred_element_type=jnp.float32)
               + lax.dot_general(M1.astype(jnp.bfloat16), tv1.astype(jnp.bfloat16),
                                 (((0,), (0,)), ((), ())),
                                 preferred_element_type=jnp.float32))
        tok_ref[pl.ds(q * CH, CH), :] = (acc[:, 0:1] * 32.0 + acc[:, 1:2]).astype(jnp.int32)
        wt_ref[pl.ds(q * CH, CH), :] = acc[:, 2:3]


def _router_dispatch(x, router_W, router_b):
    return pl.pallas_call(
        _router_dispatch_body,
        out_shape=(
            jax.ShapeDtypeStruct((P, 1), jnp.int32),
            jax.ShapeDtypeStruct((P, 1), jnp.float32),
            jax.ShapeDtypeStruct((NBLK, 1), jnp.int32),
            jax.ShapeDtypeStruct((NBLK, 1), jnp.int32),
            jax.ShapeDtypeStruct((T, 2), jnp.int32),
        ),
    )(x, router_W, router_b.reshape(1, E))


def _gelu_exact(h):
    return 0.5 * h * (1.0 + lax.erf(h * jnp.float32(0.7071067811865476)))


def _mlp_body(be_ref, act_ref, xbf_ref, tok_ref, w1_ref, b1_ref, w2_ref,
              b2_ref, wt_ref, o_ref):
    g = pl.program_id(0)

    @pl.when(act_ref[g] != 0)
    def _():
        tok_blk = tok_ref[...]
        iota_t = lax.broadcasted_iota(jnp.int32, (BLK, T), 1)
        OH = (tok_blk == iota_t).astype(jnp.float32)
        xg = _dot_default(OH, xbf_ref[...])
        h = _dot_default(xg, w1_ref[...][0]) + b1_ref[...][0]
        h = _gelu_exact(h)
        o = _dot_default(h, w2_ref[...][0]) + b2_ref[...][0]
        o_bf = o.astype(jnp.bfloat16).astype(jnp.float32)
        wt_bf = wt_ref[...].astype(jnp.bfloat16).astype(jnp.float32)
        o_ref[...] = o_bf * wt_bf


def _grouped_mlp(x, tok, w1, b1, w2, b2, wt, be, act):
    grid_spec = pltpu.PrefetchScalarGridSpec(
        num_scalar_prefetch=2,
        grid=(NBLK,),
        in_specs=[
            pl.BlockSpec((T, D), lambda g, be, act: (0, 0)),
            pl.BlockSpec((BLK, 1), lambda g, be, act: (g, 0)),
            pl.BlockSpec((1, D, H), lambda g, be, act: (be[g], 0, 0)),
            pl.BlockSpec((1, 1, H), lambda g, be, act: (be[g], 0, 0)),
            pl.BlockSpec((1, H, D), lambda g, be, act: (be[g], 0, 0)),
            pl.BlockSpec((1, 1, D), lambda g, be, act: (be[g], 0, 0)),
            pl.BlockSpec((BLK, 1), lambda g, be, act: (g, 0)),
        ],
        out_specs=pl.BlockSpec((BLK, D), lambda g, be, act: (g, 0)),
    )
    return pl.pallas_call(
        _mlp_body,
        grid_spec=grid_spec,
        out_shape=jax.ShapeDtypeStruct((P, D), jnp.float32),
    )(be, act, x, tok, w1, b1.reshape(E, 1, H), w2, b2.reshape(E, 1, D), wt)


def _sc_combine(o, posc):
    NC, NS = 2, 16
    NW = NC * NS
    t_per_w = T // NW
    CT = 16
    NCH = t_per_w // CT
    mesh = plsc.VectorSubcoreMesh(core_axis_name="c", subcore_axis_name="s")

    @functools.partial(
        pl.kernel, mesh=mesh,
        out_type=jax.ShapeDtypeStruct((T, D), jnp.float32),
        scratch_types=[
            pltpu.VMEM((NCH, 2 * CT), jnp.int32),
            pltpu.VMEM((2 * CT, D), jnp.float32),
            pltpu.VMEM((2 * CT, D), jnp.float32),
            pltpu.VMEM((CT, D), jnp.float32),
            pltpu.VMEM((CT, D), jnp.float32),
            pltpu.SemaphoreType.DMA,
            pltpu.SemaphoreType.DMA,
            pltpu.SemaphoreType.DMA,
        ],
    )
    def k(o_hbm, pc_hbm, out_hbm, idx2, rc0, rc1, ob0, ob1, s0, s1, sw):
        wid = lax.axis_index("s") * NC + lax.axis_index("c")
        base_t = wid * t_per_w
        base_i = base_t * 2
        for c in range(NCH):
            pltpu.sync_copy(pc_hbm.at[pl.ds(base_i + c * 2 * CT, 2 * CT)],
                            idx2.at[c])
        rcs = (rc0, rc1)
        obs = (ob0, ob1)
        sems = (s0, s1)
        gets = [None] * NCH
        puts = [None] * NCH
        gets[0] = pltpu.async_copy(o_hbm.at[idx2.at[0]], rc0, s0)
        for c in range(NCH):
            if c + 1 < NCH:
                gets[c + 1] = pltpu.async_copy(o_hbm.at[idx2.at[c + 1]],
                                               rcs[(c + 1) % 2],
                                               sems[(c + 1) % 2])
            gets[c].wait()
            rc = rcs[c % 2]
            ob = obs[c % 2]
            if c >= 2:
                puts[c - 2].wait()
            for r in range(CT):
                def add_vec(j, _, r=r, rc=rc, ob=ob):
                    ob[r, pl.ds(j * 16, 16)] = (rc[2 * r, pl.ds(j * 16, 16)]
                                                + rc[2 * r + 1, pl.ds(j * 16, 16)])
                    return 0
                lax.fori_loop(0, D // 16, add_vec, 0, unroll=8)
            puts[c] = pltpu.async_copy(
                ob, out_hbm.at[pl.ds(base_t + c * CT, CT)], sw)
        puts[NCH - 2].wait()
        puts[NCH - 1].wait()

    return k(o, posc)


def kernel(x, router_W, router_b, w1, b1, w2, b2):
    tok, wt, be, act, posc = _router_dispatch(x, router_W, router_b)
    o = _grouped_mlp(x, tok, w1, b1, w2, b2, wt,
                     be.reshape(NBLK), act.reshape(NBLK))
    return _sc_combine(o, posc.reshape(2 * T))
